# baseline (speedup 1.0000x reference)
import functools
import jax
import jax.numpy as jnp
from jax import lax
from jax.experimental import pallas as pl
from jax.experimental.pallas import tpu as pltpu
from jax.experimental.pallas import tpu_sc as plsc

B = 16384
H = 50
D = 64
NV = 100000
NR = 5

NC = 2
NS = 16
NW = NC * NS
NODES_PER_W = B // NW
IDX_PER_W = NODES_PER_W * H
GCHUNK = 1600
NCHUNK = IDX_PER_W // GCHUNK

_mesh = plsc.VectorSubcoreMesh(core_axis_name="c", subcore_axis_name="s")


def _prep_body(v_ref, w_ref, o_ref):
    o_ref[...] = jnp.dot(v_ref[...], w_ref[...], preferred_element_type=jnp.float32)


def _prep(v2e, wa_t):
    blk = 5000
    return pl.pallas_call(
        _prep_body,
        grid=(NV // blk,),
        in_specs=[
            pl.BlockSpec((blk, D), lambda i: (i, 0)),
            pl.BlockSpec((D, D), lambda i: (0, 0)),
        ],
        out_specs=pl.BlockSpec((blk, D), lambda i: (i, 0)),
        out_shape=jax.ShapeDtypeStruct((NV, D), jnp.float32),
    )(v2e, wa_t)


def _pad_body(huv_ref, hrr_ref, o_ref):
    blk = huv_ref.shape[0]
    z = jnp.zeros((blk, 128 - 2 * H), dtype=jnp.int32)
    o_ref[...] = jnp.concatenate(
        [huv_ref[...], z[:, : 64 - H], hrr_ref[...], z[:, : 64 - H]], axis=1)


def _pad_hist(history_uv, history_r):
    blk = 2000
    return pl.pallas_call(
        _pad_body,
        grid=(NV // blk,),
        in_specs=[
            pl.BlockSpec((blk, H), lambda i: (i, 0)),
            pl.BlockSpec((blk, H), lambda i: (i, 0)),
        ],
        out_specs=pl.BlockSpec((blk, 128), lambda i: (i, 0)),
        out_shape=jax.ShapeDtypeStruct((NV, 128), jnp.int32),
    )(history_uv, history_r)


@functools.partial(
    pl.kernel,
    mesh=_mesh,
    compiler_params=pltpu.CompilerParams(use_tc_tiling_on_sc=False),
    out_type=(
        jax.ShapeDtypeStruct((B, 128), jnp.int32),
        jax.ShapeDtypeStruct((B, D), jnp.float32),
    ),
    scratch_types=[
        pltpu.VMEM((NODES_PER_W,), jnp.int32),
        pltpu.VMEM((NODES_PER_W, 128), jnp.int32),
        pltpu.VMEM((NODES_PER_W, D), jnp.float32),
        pltpu.SemaphoreType.DMA,
    ],
)
def _sc_gather_nodes(nodes_hbm, hist_hbm, u2e_hbm,
                     hist_out, uv_out,
                     idx_v, hist_v, uv_v, sem):
    wid = lax.axis_index("s") * NC + lax.axis_index("c")
    base = wid * NODES_PER_W
    pltpu.sync_copy(nodes_hbm.at[pl.ds(base, NODES_PER_W)], idx_v)
    a = pltpu.async_copy(hist_hbm.at[idx_v], hist_v, sem)
    b = pltpu.async_copy(u2e_hbm.at[idx_v], uv_v, sem)
    a.wait()
    b.wait()
    pltpu.sync_copy(hist_v, hist_out.at[pl.ds(base, NODES_PER_W)])
    pltpu.sync_copy(uv_v, uv_out.at[pl.ds(base, NODES_PER_W)])


@functools.partial(
    pl.kernel,
    mesh=_mesh,
    compiler_params=pltpu.CompilerParams(use_tc_tiling_on_sc=False),
    out_type=jax.ShapeDtypeStruct((B * H, D), jnp.float32),
    scratch_types=[
        pltpu.VMEM((GCHUNK,), jnp.int32),
        pltpu.VMEM((GCHUNK, D), jnp.float32),
        pltpu.SemaphoreType.DMA,
    ],
)
def _sc_gather_items(vtab_hbm, idx_hbm, out_hbm, idx_v, rows_v, sem):
    wid = lax.axis_index("s") * NC + lax.axis_index("c")

    def body(c, carry):
        base = wid * IDX_PER_W + c * GCHUNK
        pltpu.sync_copy(idx_hbm.at[pl.ds(base, GCHUNK)], idx_v)
        pltpu.async_copy(vtab_hbm.at[idx_v], rows_v, sem).wait()
        pltpu.sync_copy(rows_v, out_hbm.at[pl.ds(base, GCHUNK)])
        return carry

    lax.fori_loop(0, NCHUNK, body, 0)


def _main_body(vg_ref, hr_ref, uv_ref, r2e_ref, w1b_t_ref, w_r1_b_ref,
               w2_t_ref, w_r2_b_ref, a1o_t_ref, a1u_t_ref, att1_b_ref,
               a2_t_ref, att2_b_ref, att3_ref, l1u_t_ref, l1n_t_ref,
               lin1_b_ref, o_ref,
               up_s, rproj_s, macc, sacc, nacc, *, nblk):
    h = pl.program_id(1)
    uv = uv_ref[...]

    @pl.when(h == 0)
    def _init():
        up_s[...] = jnp.dot(uv, a1u_t_ref[...],
                            preferred_element_type=jnp.float32)
        rp = jnp.dot(r2e_ref[...], w1b_t_ref[...],
                     preferred_element_type=jnp.float32) + w_r1_b_ref[...]
        rproj_s[...] = jnp.concatenate(
            [rp, jnp.zeros((8 - NR, D), jnp.float32)], axis=0)
        macc[...] = jnp.full((nblk, 1), -1e30, jnp.float32)
        sacc[...] = jnp.zeros((nblk, 1), jnp.float32)
        nacc[...] = jnp.zeros((nblk, D), jnp.float32)

    r = hr_ref[...].reshape(1, nblk)
    onehot_t = (lax.broadcasted_iota(jnp.int32, (8, nblk), 0)
                == jnp.broadcast_to(r, (8, nblk))).astype(jnp.float32)
    er = lax.dot_general(onehot_t, rproj_s[...],
                         (((0,), (0,)), ((), ())),
                         preferred_element_type=jnp.float32)

    x1 = jnp.maximum(vg_ref[...] + er, 0.0)
    o = jnp.maximum(jnp.dot(x1, w2_t_ref[...],
                            preferred_element_type=jnp.float32)
                    + w_r2_b_ref[...], 0.0)
    a1 = jnp.maximum(jnp.dot(o, a1o_t_ref[...],
                             preferred_element_type=jnp.float32)
                     + up_s[...] + att1_b_ref[...], 0.0)
    a2 = jnp.maximum(jnp.dot(a1, a2_t_ref[...],
                             preferred_element_type=jnp.float32)
                     + att2_b_ref[...], 0.0)
    l = jnp.sum(a2 * att3_ref[...], axis=1, keepdims=True)

    m_new = jnp.maximum(macc[...], l)
    c = jnp.exp(macc[...] - m_new)
    e = jnp.exp(l - m_new)
    sacc[...] = sacc[...] * c + e
    nacc[...] = nacc[...] * c + o * e
    macc[...] = m_new

    @pl.when(h == H - 1)
    def _fin():
        neigh = nacc[...] / sacc[...]
        o_ref[...] = jnp.maximum(
            jnp.dot(uv, l1u_t_ref[...], preferred_element_type=jnp.float32)
            + jnp.dot(neigh, l1n_t_ref[...], preferred_element_type=jnp.float32)
            + lin1_b_ref[...], 0.0)


def _main(vg, hr3, uv, r2e, w1b_t, w_r1_b, w2_t, w_r2_b,
          a1o_t, a1u_t, att1_b, a2_t, att2_b, att3, l1u_t, l1n_t, lin1_b):
    nblk = 2048
    nb = B // nblk
    grid = (nb, H)
    full = lambda shape: pl.BlockSpec(shape, lambda i, h: tuple(0 for _ in shape))
    return pl.pallas_call(
        functools.partial(_main_body, nblk=nblk),
        grid=grid,
        in_specs=[
            pl.BlockSpec((nblk, D), lambda i, h: (h * nb + i, 0)),
            pl.BlockSpec((1, 1, nblk), lambda i, h: (h, 0, i)),
            pl.BlockSpec((nblk, D), lambda i, h: (i, 0)),
            full((NR, D)), full((D, D)), full((D,)),
            full((D, D)), full((D,)),
            full((D, D)), full((D, D)), full((D,)),
            full((D, D)), full((D,)),
            full((D,)),
            full((D, D)), full((D, D)), full((D,)),
        ],
        out_specs=pl.BlockSpec((nblk, D), lambda i, h: (i, 0)),
        out_shape=jax.ShapeDtypeStruct((B, D), jnp.float32),
        scratch_shapes=[
            pltpu.VMEM((nblk, D), jnp.float32),
            pltpu.VMEM((8, D), jnp.float32),
            pltpu.VMEM((nblk, 1), jnp.float32),
            pltpu.VMEM((nblk, 1), jnp.float32),
            pltpu.VMEM((nblk, D), jnp.float32),
        ],
    )(vg, hr3, uv, r2e, w1b_t, w_r1_b, w2_t, w_r2_b,
      a1o_t, a1u_t, att1_b, a2_t, att2_b, att3, l1u_t, l1n_t, lin1_b)


def kernel(nodes, history_uv, history_r, u2e, v2e, r2e,
           w_r1_w, w_r1_b, w_r2_w, w_r2_b,
           att1_w, att1_b, att2_w, att2_b, att3_w, att3_b,
           lin1_w, lin1_b):
    wa_t = jnp.transpose(w_r1_w[:, :D])
    w1b_t = jnp.transpose(w_r1_w[:, D:])
    w2_t = jnp.transpose(w_r2_w)
    a1o_t = jnp.transpose(att1_w[:, :D])
    a1u_t = jnp.transpose(att1_w[:, D:])
    a2_t = jnp.transpose(att2_w)
    att3 = att3_w[0]
    l1u_t = jnp.transpose(lin1_w[:, :D])
    l1n_t = jnp.transpose(lin1_w[:, D:])

    vproj = _prep(v2e, wa_t)
    hist_pad = _pad_hist(history_uv, history_r)
    hist_g, uv = _sc_gather_nodes(nodes, hist_pad, u2e)
    hi = hist_g[:, :H]
    hr = hist_g[:, 64:64 + H]
    idx = jnp.transpose(hi).reshape(-1)
    vg = _sc_gather_items(vproj, idx)
    hr3 = jnp.transpose(hr).reshape(H, 1, B)
    return _main(vg, hr3, uv, r2e, w1b_t, w_r1_b, w2_t, w_r2_b,
                 a1o_t, a1u_t, att1_b, a2_t, att2_b, att3, l1u_t, l1n_t, lin1_b)


# speedup vs baseline: 3.0212x; 3.0212x over previous
import functools
import jax
import jax.numpy as jnp
from jax import lax
from jax.experimental import pallas as pl
from jax.experimental.pallas import tpu as pltpu
from jax.experimental.pallas import tpu_sc as plsc

B = 16384
H = 50
D = 64
NV = 100000
NR = 5

NC = 2
NS = 16
NW = NC * NS
NODES_PER_W = B // NW
IDX_PER_W = NODES_PER_W * H
GCHUNK = 1600
NCHUNK = IDX_PER_W // GCHUNK

_mesh = plsc.VectorSubcoreMesh(core_axis_name="c", subcore_axis_name="s")


def _mktable_body(vp_ref, wbd_ref, r2e_ref, w1bt_ref, w1b_ref, o_ref):
    rp = jnp.dot(r2e_ref[0], w1bt_ref[...],
                 preferred_element_type=jnp.float32) + w1b_ref[...]
    rp2 = jnp.concatenate([rp, rp], axis=1)
    o_ref[...] = jnp.dot(vp_ref[...], wbd_ref[...],
                         preferred_element_type=jnp.float32) + rp2


def _mktable(v2e_pair, wa_bd, r2e3, w1b_t, w_r1_b):
    blk = 2000
    nc = (NV // 2) // blk
    return pl.pallas_call(
        _mktable_body,
        grid=(nc, NR),
        in_specs=[
            pl.BlockSpec((blk, 2 * D), lambda c, r: (c, 0)),
            pl.BlockSpec((2 * D, 2 * D), lambda c, r: (0, 0)),
            pl.BlockSpec((1, 1, D), lambda c, r: (r, 0, 0)),
            pl.BlockSpec((D, D), lambda c, r: (0, 0)),
            pl.BlockSpec((D,), lambda c, r: (0,)),
        ],
        out_specs=pl.BlockSpec((blk, 2 * D), lambda c, r: (r * nc + c, 0)),
        out_shape=jax.ShapeDtypeStruct((NR * NV // 2, 2 * D), jnp.float32),
    )(v2e_pair, wa_bd, r2e3, w1b_t, w_r1_b)


def _pad_body(huv_ref, hrr_ref, o_ref):
    blk = huv_ref.shape[0]
    z = jnp.zeros((blk, 128 - 2 * H), dtype=jnp.int32)
    o_ref[...] = jnp.concatenate(
        [huv_ref[...], z[:, : 64 - H], hrr_ref[...], z[:, : 64 - H]], axis=1)


def _pad_hist(history_uv, history_r):
    blk = 2000
    return pl.pallas_call(
        _pad_body,
        grid=(NV // blk,),
        in_specs=[
            pl.BlockSpec((blk, H), lambda i: (i, 0)),
            pl.BlockSpec((blk, H), lambda i: (i, 0)),
        ],
        out_specs=pl.BlockSpec((blk, 128), lambda i: (i, 0)),
        out_shape=jax.ShapeDtypeStruct((NV, 128), jnp.int32),
    )(history_uv, history_r)


@functools.partial(
    pl.kernel,
    mesh=_mesh,
    compiler_params=pltpu.CompilerParams(use_tc_tiling_on_sc=False),
    out_type=(
        jax.ShapeDtypeStruct((B, 128), jnp.int32),
        jax.ShapeDtypeStruct((B, D), jnp.float32),
    ),
    scratch_types=[
        pltpu.VMEM((NODES_PER_W,), jnp.int32),
        pltpu.VMEM((NODES_PER_W, 128), jnp.int32),
        pltpu.VMEM((NODES_PER_W, D), jnp.float32),
        pltpu.SemaphoreType.DMA,
    ],
)
def _sc_gather_nodes(nodes_hbm, hist_hbm, u2e_hbm,
                     hist_out, uv_out,
                     idx_v, hist_v, uv_v, sem):
    wid = lax.axis_index("s") * NC + lax.axis_index("c")
    base = wid * NODES_PER_W
    pltpu.sync_copy(nodes_hbm.at[pl.ds(base, NODES_PER_W)], idx_v)
    a = pltpu.async_copy(hist_hbm.at[idx_v], hist_v, sem)
    b = pltpu.async_copy(u2e_hbm.at[idx_v], uv_v, sem)
    a.wait()
    b.wait()
    pltpu.sync_copy(hist_v, hist_out.at[pl.ds(base, NODES_PER_W)])
    pltpu.sync_copy(uv_v, uv_out.at[pl.ds(base, NODES_PER_W)])


@functools.partial(
    pl.kernel,
    mesh=_mesh,
    compiler_params=pltpu.CompilerParams(use_tc_tiling_on_sc=False),
    out_type=jax.ShapeDtypeStruct((B * H, D), jnp.float32),
    scratch_types=[
        pltpu.VMEM((GCHUNK,), jnp.int32),
        pltpu.VMEM((GCHUNK, D), jnp.float32),
        pltpu.SemaphoreType.DMA,
    ],
)
def _sc_gather_items(vtab_hbm, idx_hbm, out_hbm, idx_v, rows_v, sem):
    wid = lax.axis_index("s") * NC + lax.axis_index("c")

    def body(c, carry):
        base = wid * IDX_PER_W + c * GCHUNK
        pltpu.sync_copy(idx_hbm.at[pl.ds(base, GCHUNK)], idx_v)
        pltpu.async_copy(vtab_hbm.at[idx_v], rows_v, sem).wait()
        pltpu.sync_copy(rows_v, out_hbm.at[pl.ds(base, GCHUNK)])
        return carry

    lax.fori_loop(0, NCHUNK, body, 0)


_NPAIR = 128
_MROWS = H * _NPAIR


def _main_body(vg_ref, uv_ref, wbd2_ref, b2_ref, abd1o_ref, abd1u_ref,
               a1b_ref, abd2_ref, a2b_ref, att3m_ref, lbd1u_ref, lbd1n_ref,
               l1b_ref, o_ref):
    x1 = jnp.maximum(vg_ref[...], 0.0)
    o = jnp.maximum(jnp.dot(x1, wbd2_ref[...],
                            preferred_element_type=jnp.float32)
                    + b2_ref[...], 0.0)
    uvp = uv_ref[...]
    up = jnp.dot(uvp, abd1u_ref[...], preferred_element_type=jnp.float32)
    upb = jnp.broadcast_to(up[None], (H, _NPAIR, 2 * D)).reshape(_MROWS, 2 * D)
    a1 = jnp.maximum(jnp.dot(o, abd1o_ref[...],
                             preferred_element_type=jnp.float32)
                     + upb + a1b_ref[...], 0.0)
    a2 = jnp.maximum(jnp.dot(a1, abd2_ref[...],
                             preferred_element_type=jnp.float32)
                     + a2b_ref[...], 0.0)
    l128 = jnp.dot(a2, att3m_ref[...], preferred_element_type=jnp.float32)
    l3 = l128.reshape(H, _NPAIR, 2 * D)
    m = jnp.max(l3, axis=0, keepdims=True)
    e3 = jnp.exp(l3 - m)
    s = jnp.sum(e3, axis=0, keepdims=True)
    w3 = e3 / s
    o3 = o.reshape(H, _NPAIR, 2 * D)
    neigh = jnp.sum(o3 * w3, axis=0)
    o_ref[...] = jnp.maximum(
        jnp.dot(uvp, lbd1u_ref[...], preferred_element_type=jnp.float32)
        + jnp.dot(neigh, lbd1n_ref[...], preferred_element_type=jnp.float32)
        + l1b_ref[...], 0.0)


def _main(vgp, uvp, wbd2, b2, abd1o, abd1u, a1b, abd2, a2b, att3m,
          lbd1u, lbd1n, l1b):
    nb = B // (2 * _NPAIR)
    full = lambda shape: pl.BlockSpec(shape, lambda i: tuple(0 for _ in shape))
    return pl.pallas_call(
        _main_body,
        grid=(nb,),
        in_specs=[
            pl.BlockSpec((_MROWS, 2 * D), lambda i: (i, 0)),
            pl.BlockSpec((_NPAIR, 2 * D), lambda i: (i, 0)),
            full((2 * D, 2 * D)), full((2 * D,)),
            full((2 * D, 2 * D)), full((2 * D, 2 * D)), full((2 * D,)),
            full((2 * D, 2 * D)), full((2 * D,)),
            full((2 * D, 2 * D)),
            full((2 * D, 2 * D)), full((2 * D, 2 * D)), full((2 * D,)),
        ],
        out_specs=pl.BlockSpec((_NPAIR, 2 * D), lambda i: (i, 0)),
        out_shape=jax.ShapeDtypeStruct((B // 2, 2 * D), jnp.float32),
    )(vgp, uvp, wbd2, b2, abd1o, abd1u, a1b, abd2, a2b, att3m,
      lbd1u, lbd1n, l1b)


def _bd(m):
    z = jnp.zeros_like(m)
    return jnp.concatenate(
        [jnp.concatenate([m, z], axis=1), jnp.concatenate([z, m], axis=1)],
        axis=0)


def kernel(nodes, history_uv, history_r, u2e, v2e, r2e,
           w_r1_w, w_r1_b, w_r2_w, w_r2_b,
           att1_w, att1_b, att2_w, att2_b, att3_w, att3_b,
           lin1_w, lin1_b):
    wa_t = jnp.transpose(w_r1_w[:, :D])
    w1b_t = jnp.transpose(w_r1_w[:, D:])
    wbd2 = _bd(jnp.transpose(w_r2_w))
    abd1o = _bd(jnp.transpose(att1_w[:, :D]))
    abd1u = _bd(jnp.transpose(att1_w[:, D:]))
    abd2 = _bd(jnp.transpose(att2_w))
    att3m = _bd(att3_w[0][:, None] * jnp.ones((1, D), jnp.float32))
    lbd1u = _bd(jnp.transpose(lin1_w[:, :D]))
    lbd1n = _bd(jnp.transpose(lin1_w[:, D:]))
    b2 = jnp.concatenate([w_r2_b, w_r2_b])
    a1b = jnp.concatenate([att1_b, att1_b])
    a2b = jnp.concatenate([att2_b, att2_b])
    l1b = jnp.concatenate([lin1_b, lin1_b])

    tbl_pair = _mktable(v2e.reshape(NV // 2, 2 * D), _bd(wa_t),
                        r2e[:, None, :], w1b_t, w_r1_b)
    tbl = tbl_pair.reshape(NR * NV, D)

    hist_pad = _pad_hist(history_uv, history_r)
    hist_g, uv = _sc_gather_nodes(nodes, hist_pad, u2e)
    hi = hist_g[:, :H]
    hr = hist_g[:, 64:64 + H]
    idxc = hi + NV * hr
    idx = jnp.transpose(idxc.reshape(B // (2 * _NPAIR), 2 * _NPAIR, H),
                        (0, 2, 1)).reshape(-1)
    vg = _sc_gather_items(tbl, idx)
    vgp = vg.reshape(B * H // 2, 2 * D)
    uvp = uv.reshape(B // 2, 2 * D)
    outp = _main(vgp, uvp, wbd2, b2, abd1o, abd1u, a1b, abd2, a2b, att3m,
                 lbd1u, lbd1n, l1b)
    return outp.reshape(B, D)


# speedup vs baseline: 3.2119x; 1.0631x over previous
import functools
import jax
import jax.numpy as jnp
from jax import lax
from jax.experimental import pallas as pl
from jax.experimental.pallas import tpu as pltpu
from jax.experimental.pallas import tpu_sc as plsc

B = 16384
H = 50
D = 64
NV = 100000
NR = 5

NC = 2
NS = 16
NW = NC * NS
NODES_PER_W = B // NW
GCHUNK = 800

_mesh = plsc.VectorSubcoreMesh(core_axis_name="c", subcore_axis_name="s")


def _mktable_body(vp_ref, wbd_ref, r2e_ref, w1bt_ref, w1b_ref, o_ref):
    rp = jnp.dot(r2e_ref[0], w1bt_ref[...],
                 preferred_element_type=jnp.float32) + w1b_ref[...]
    rp2 = jnp.concatenate([rp, rp], axis=1)
    o_ref[...] = jnp.dot(vp_ref[...], wbd_ref[...],
                         preferred_element_type=jnp.float32) + rp2


def _mktable(v2e_pair, wa_bd, r2e3, w1b_t, w_r1_b):
    blk = 2000
    nc = (NV // 2) // blk
    return pl.pallas_call(
        _mktable_body,
        grid=(nc, NR),
        in_specs=[
            pl.BlockSpec((blk, 2 * D), lambda c, r: (c, 0)),
            pl.BlockSpec((2 * D, 2 * D), lambda c, r: (0, 0)),
            pl.BlockSpec((1, 1, D), lambda c, r: (r, 0, 0)),
            pl.BlockSpec((D, D), lambda c, r: (0, 0)),
            pl.BlockSpec((D,), lambda c, r: (0,)),
        ],
        out_specs=pl.BlockSpec((blk, 2 * D), lambda c, r: (r * nc + c, 0)),
        out_shape=jax.ShapeDtypeStruct((NR * NV // 2, 2 * D), jnp.float32),
    )(v2e_pair, wa_bd, r2e3, w1b_t, w_r1_b)


def _pad_body(huv_ref, hrr_ref, o_ref):
    blk = huv_ref.shape[0]
    z = jnp.zeros((blk, 128 - 2 * H), dtype=jnp.int32)
    o_ref[...] = jnp.concatenate(
        [huv_ref[...], z[:, : 64 - H], hrr_ref[...], z[:, : 64 - H]], axis=1)


def _pad_hist(history_uv, history_r):
    blk = 2000
    return pl.pallas_call(
        _pad_body,
        grid=(NV // blk,),
        in_specs=[
            pl.BlockSpec((blk, H), lambda i: (i, 0)),
            pl.BlockSpec((blk, H), lambda i: (i, 0)),
        ],
        out_specs=pl.BlockSpec((blk, 128), lambda i: (i, 0)),
        out_shape=jax.ShapeDtypeStruct((NV, 128), jnp.int32),
    )(history_uv, history_r)


@functools.partial(
    pl.kernel,
    mesh=_mesh,
    compiler_params=pltpu.CompilerParams(use_tc_tiling_on_sc=False),
    out_type=(
        jax.ShapeDtypeStruct((B, 128), jnp.int32),
        jax.ShapeDtypeStruct((B, D), jnp.float32),
    ),
    scratch_types=[
        pltpu.VMEM((NODES_PER_W,), jnp.int32),
        pltpu.VMEM((NODES_PER_W, 128), jnp.int32),
        pltpu.VMEM((NODES_PER_W, D), jnp.float32),
        pltpu.SemaphoreType.DMA,
    ],
)
def _sc_gather_nodes(nodes_hbm, hist_hbm, u2e_hbm,
                     hist_out, uv_out,
                     idx_v, hist_v, uv_v, sem):
    wid = lax.axis_index("s") * NC + lax.axis_index("c")
    base = wid * NODES_PER_W
    pltpu.sync_copy(nodes_hbm.at[pl.ds(base, NODES_PER_W)], idx_v)
    a = pltpu.async_copy(hist_hbm.at[idx_v], hist_v, sem)
    b = pltpu.async_copy(u2e_hbm.at[idx_v], uv_v, sem)
    a.wait()
    b.wait()
    pltpu.sync_copy(hist_v, hist_out.at[pl.ds(base, NODES_PER_W)])
    pltpu.sync_copy(uv_v, uv_out.at[pl.ds(base, NODES_PER_W)])


def _make_sc_gather(nidx):
    per_w = nidx // NW
    nch = per_w // GCHUNK

    @functools.partial(
        pl.kernel,
        mesh=_mesh,
        compiler_params=pltpu.CompilerParams(use_tc_tiling_on_sc=False),
        out_type=jax.ShapeDtypeStruct((nidx, D), jnp.float32),
        scratch_types=[
            pltpu.VMEM((GCHUNK,), jnp.int32),
            pltpu.VMEM((GCHUNK,), jnp.int32),
            pltpu.VMEM((GCHUNK, D), jnp.float32),
            pltpu.VMEM((GCHUNK, D), jnp.float32),
            pltpu.SemaphoreType.DMA,
            pltpu.SemaphoreType.DMA,
        ],
    )
    def k(vtab_hbm, idx_hbm, out_hbm, idx0, idx1, r0, r1, s0, s1):
        wid = lax.axis_index("s") * NC + lax.axis_index("c")
        base = wid * per_w
        idxv = [idx0, idx1]
        rv = [r0, r1]
        sv = [s0, s1]
        h = [None, None]
        pltpu.sync_copy(idx_hbm.at[pl.ds(base, GCHUNK)], idx0)
        h[0] = pltpu.async_copy(vtab_hbm.at[idx0], r0, s0)
        for c in range(nch):
            cur = c & 1
            nxt = 1 - cur
            if c + 1 < nch:
                pltpu.sync_copy(
                    idx_hbm.at[pl.ds(base + (c + 1) * GCHUNK, GCHUNK)],
                    idxv[nxt])
                h[nxt] = pltpu.async_copy(vtab_hbm.at[idxv[nxt]], rv[nxt],
                                          sv[nxt])
            h[cur].wait()
            pltpu.sync_copy(rv[cur], out_hbm.at[pl.ds(base + c * GCHUNK,
                                                      GCHUNK)])

    return k


_sc_gather_half = _make_sc_gather(B * H // 2)


_NPAIR = 128
_MROWS = H * _NPAIR


def _main_body(vg_ref, uv_ref, wbd2_ref, b2_ref, abd1o_ref, abd1u_ref,
               a1b_ref, abd2_ref, a2b_ref, att3m_ref, lbd1u_ref, lbd1n_ref,
               l1b_ref, o_ref):
    x1 = jnp.maximum(vg_ref[...], 0.0)
    o = jnp.maximum(jnp.dot(x1, wbd2_ref[...],
                            preferred_element_type=jnp.float32)
                    + b2_ref[...], 0.0)
    uvp = uv_ref[...]
    up = jnp.dot(uvp, abd1u_ref[...], preferred_element_type=jnp.float32)
    upb = jnp.broadcast_to(up[None], (H, _NPAIR, 2 * D)).reshape(_MROWS, 2 * D)
    a1 = jnp.maximum(jnp.dot(o, abd1o_ref[...],
                             preferred_element_type=jnp.float32)
                     + upb + a1b_ref[...], 0.0)
    a2 = jnp.maximum(jnp.dot(a1, abd2_ref[...],
                             preferred_element_type=jnp.float32)
                     + a2b_ref[...], 0.0)
    l128 = jnp.dot(a2, att3m_ref[...], preferred_element_type=jnp.float32)
    l3 = l128.reshape(H, _NPAIR, 2 * D)
    m = jnp.max(l3, axis=0, keepdims=True)
    e3 = jnp.exp(l3 - m)
    s = jnp.sum(e3, axis=0, keepdims=True)
    w3 = e3 / s
    o3 = o.reshape(H, _NPAIR, 2 * D)
    neigh = jnp.sum(o3 * w3, axis=0)
    o_ref[...] = jnp.maximum(
        jnp.dot(uvp, lbd1u_ref[...], preferred_element_type=jnp.float32)
        + jnp.dot(neigh, lbd1n_ref[...], preferred_element_type=jnp.float32)
        + l1b_ref[...], 0.0)


def _main(vgp, uvp, wbd2, b2, abd1o, abd1u, a1b, abd2, a2b, att3m,
          lbd1u, lbd1n, l1b):
    nb = vgp.shape[0] // _MROWS
    full = lambda shape: pl.BlockSpec(shape, lambda i: tuple(0 for _ in shape))
    return pl.pallas_call(
        _main_body,
        grid=(nb,),
        in_specs=[
            pl.BlockSpec((_MROWS, 2 * D), lambda i: (i, 0)),
            pl.BlockSpec((_NPAIR, 2 * D), lambda i: (i, 0)),
            full((2 * D, 2 * D)), full((2 * D,)),
            full((2 * D, 2 * D)), full((2 * D, 2 * D)), full((2 * D,)),
            full((2 * D, 2 * D)), full((2 * D,)),
            full((2 * D, 2 * D)),
            full((2 * D, 2 * D)), full((2 * D, 2 * D)), full((2 * D,)),
        ],
        out_specs=pl.BlockSpec((_NPAIR, 2 * D), lambda i: (i, 0)),
        out_shape=jax.ShapeDtypeStruct((nb * _NPAIR, 2 * D), jnp.float32),
    )(vgp, uvp, wbd2, b2, abd1o, abd1u, a1b, abd2, a2b, att3m,
      lbd1u, lbd1n, l1b)


def _bd(m):
    z = jnp.zeros_like(m)
    return jnp.concatenate(
        [jnp.concatenate([m, z], axis=1), jnp.concatenate([z, m], axis=1)],
        axis=0)


def kernel(nodes, history_uv, history_r, u2e, v2e, r2e,
           w_r1_w, w_r1_b, w_r2_w, w_r2_b,
           att1_w, att1_b, att2_w, att2_b, att3_w, att3_b,
           lin1_w, lin1_b):
    wa_t = jnp.transpose(w_r1_w[:, :D])
    w1b_t = jnp.transpose(w_r1_w[:, D:])
    wbd2 = _bd(jnp.transpose(w_r2_w))
    abd1o = _bd(jnp.transpose(att1_w[:, :D]))
    abd1u = _bd(jnp.transpose(att1_w[:, D:]))
    abd2 = _bd(jnp.transpose(att2_w))
    att3m = _bd(att3_w[0][:, None] * jnp.ones((1, D), jnp.float32))
    lbd1u = _bd(jnp.transpose(lin1_w[:, :D]))
    lbd1n = _bd(jnp.transpose(lin1_w[:, D:]))
    b2 = jnp.concatenate([w_r2_b, w_r2_b])
    a1b = jnp.concatenate([att1_b, att1_b])
    a2b = jnp.concatenate([att2_b, att2_b])
    l1b = jnp.concatenate([lin1_b, lin1_b])

    tbl_pair = _mktable(v2e.reshape(NV // 2, 2 * D), _bd(wa_t),
                        r2e[:, None, :], w1b_t, w_r1_b)
    tbl = tbl_pair.reshape(NR * NV, D)

    hist_pad = _pad_hist(history_uv, history_r)
    hist_g, uv = _sc_gather_nodes(nodes, hist_pad, u2e)
    hi = hist_g[:, :H]
    hr = hist_g[:, 64:64 + H]
    idxc = hi + NV * hr
    idx = jnp.transpose(idxc.reshape(B // (2 * _NPAIR), 2 * _NPAIR, H),
                        (0, 2, 1)).reshape(-1)
    half = B * H // 2
    uvp = uv.reshape(B // 2, 2 * D)
    outs = []
    for part in range(2):
        vg = _sc_gather_half(tbl, idx[part * half:(part + 1) * half])
        vgp = vg.reshape(half // 2, 2 * D)
        uvp_p = uvp[part * (B // 4):(part + 1) * (B // 4)]
        outs.append(_main(vgp, uvp_p, wbd2, b2, abd1o, abd1u, a1b, abd2,
                          a2b, att3m, lbd1u, lbd1n, l1b))
    return jnp.concatenate(outs, axis=0).reshape(B, D)
